# Initial kernel scaffold; baseline (speedup 1.0000x reference)
import jax
import jax.numpy as jnp
from jax.experimental import pallas as pl


def kernel(x, W, Wz, Uz, Wr, Ur, Wh, Uh, edge_index):
    raise NotImplementedError("write your pallas kernel here")


# speedup vs baseline: 4.8087x; 4.8087x over previous
import functools

import jax
import jax.numpy as jnp
from jax import lax
from jax.experimental import pallas as pl
from jax.experimental.pallas import tpu as pltpu
from jax.experimental.pallas import tpu_sc as plsc

_N = 10000
_D = 128
_E = 320000
_L = 3

_NP = 10240
_NW = 32
_EPW = _E // _NW
_CH = 80
_NCH = _EPW // _CH
_RPT = _NP // 16

_mesh = plsc.VectorSubcoreMesh(core_axis_name="c", subcore_axis_name="s")


@functools.partial(
    pl.kernel,
    mesh=_mesh,
    out_type=jax.ShapeDtypeStruct((2 * _NP, _D), jnp.float32),
    scratch_types=[
        pltpu.VMEM((_CH,), jnp.int32),
        pltpu.VMEM((_CH,), jnp.int32),
        pltpu.VMEM((_CH, _D), jnp.float32),
        pltpu.VMEM_SHARED((_NP, _D), jnp.float32),
        pltpu.SemaphoreType.DMA,
    ],
)
def _segsum(h_hbm, src_hbm, dst_hbm, zeros_hbm, out_hbm,
            src_v, dst_v, rows_v, acc, sem):
    c = lax.axis_index("c")
    s = lax.axis_index("s")
    wid = c * 16 + s

    pltpu.sync_copy(zeros_hbm, acc.at[pl.ds(s * _RPT, _RPT)])
    plsc.subcore_barrier()

    base = wid * _NCH

    def body(j, carry):
        pltpu.sync_copy(src_hbm.at[base + j], src_v)
        pltpu.sync_copy(dst_hbm.at[base + j], dst_v)
        pltpu.async_copy(h_hbm.at[src_v], rows_v, sem).wait()
        pltpu.sync_copy(rows_v, acc.at[dst_v], add=True)
        return carry

    lax.fori_loop(0, _NCH, body, 0)

    plsc.subcore_barrier()
    pltpu.sync_copy(acc.at[pl.ds(s * _RPT, _RPT)],
                    out_hbm.at[pl.ds(c * _NP + s * _RPT, _RPT)])


_BLK = 1024


def _gru_body(g0_ref, g1_ref, h_ref, w_ref, wz_ref, uz_ref, wr_ref, ur_ref,
              wh_ref, uh_ref, out_ref):
    f32 = jnp.float32
    g = g0_ref[...] + g1_ref[...]
    h = h_ref[...]
    agg = jnp.dot(g, w_ref[...], preferred_element_type=f32)
    z = jax.nn.sigmoid(jnp.dot(agg, wz_ref[...], preferred_element_type=f32)
                       + jnp.dot(h, uz_ref[...], preferred_element_type=f32))
    r = jax.nn.sigmoid(jnp.dot(agg, wr_ref[...], preferred_element_type=f32)
                       + jnp.dot(h, ur_ref[...], preferred_element_type=f32))
    hh = jnp.tanh(jnp.dot(agg, wh_ref[...], preferred_element_type=f32)
                  + jnp.dot(r * h, uh_ref[...], preferred_element_type=f32))
    out_ref[...] = (1.0 - z) * h + z * hh


_row_spec = pl.BlockSpec((_BLK, _D), lambda i: (i, 0))
_w_spec = pl.BlockSpec((_D, _D), lambda i: (0, 0))

_gru = pl.pallas_call(
    _gru_body,
    grid=(_NP // _BLK,),
    in_specs=[_row_spec, _row_spec, _row_spec] + [_w_spec] * 7,
    out_specs=_row_spec,
    out_shape=jax.ShapeDtypeStruct((_NP, _D), jnp.float32),
)


def kernel(x, W, Wz, Uz, Wr, Ur, Wh, Uh, edge_index):
    src = edge_index[0].astype(jnp.int32).reshape(_E // _CH, _CH)
    dst = edge_index[1].astype(jnp.int32).reshape(_E // _CH, _CH)
    zeros = jnp.zeros((_RPT, _D), jnp.float32)
    h = jnp.pad(x, ((0, _NP - _N), (0, 0)))
    for l in range(_L):
        g2 = _segsum(h, src, dst, zeros)
        h = _gru(g2[:_NP], g2[_NP:], h, W[l], Wz, Uz, Wr, Ur, Wh, Uh)
    return jnp.concatenate([x, h[:_N]], axis=-1)


# speedup vs baseline: 9.4110x; 1.9571x over previous
import functools

import jax
import jax.numpy as jnp
from jax import lax
from jax.experimental import pallas as pl
from jax.experimental.pallas import tpu as pltpu
from jax.experimental.pallas import tpu_sc as plsc

_N = 10000
_D = 128
_E = 320000
_L = 3

_NP = 10240
_NW = 32
_EPW = _E // _NW
_CH = 125
_NCH = _EPW // _CH
_RB = 2
_NG = _NCH // _RB
_RPT = _NP // 16

_mesh = plsc.VectorSubcoreMesh(core_axis_name="c", subcore_axis_name="s")


@functools.partial(
    pl.kernel,
    mesh=_mesh,
    out_type=jax.ShapeDtypeStruct((2 * _NP, _D), jnp.float32),
    scratch_types=[
        pltpu.VMEM_SHARED((_NP, _D), jnp.float32),
        pltpu.VMEM((_NCH, _CH), jnp.int32),
    ]
    + [pltpu.VMEM((1, _CH), jnp.int32)] * _RB
    + [pltpu.VMEM((_CH, _D), jnp.float32)] * _RB
    + [pltpu.SemaphoreType.DMA] * (3 * _RB),
)
def _segsum(h_hbm, src_hbm, dst_hbm, zeros_hbm, out_hbm,
            acc, dst_v, *bufs_and_sems):
    srcb = bufs_and_sems[0 * _RB:1 * _RB]
    rows = bufs_and_sems[1 * _RB:2 * _RB]
    isem = bufs_and_sems[2 * _RB:3 * _RB]
    gsem = bufs_and_sems[3 * _RB:4 * _RB]
    ssem = bufs_and_sems[4 * _RB:]
    c = lax.axis_index("c")
    s = lax.axis_index("s")
    wid = c * 16 + s
    base = wid * _NCH

    def fire_idx(j, b):
        pltpu.async_copy(src_hbm.at[pl.ds(base + j, 1)], srcb[b], isem[b])

    def wait_idx(j, b):
        pltpu.make_async_copy(src_hbm.at[pl.ds(base + j, 1)], srcb[b],
                              isem[b]).wait()

    for b in range(_RB):
        fire_idx(b, b)
    pltpu.sync_copy(dst_hbm.at[pl.ds(base, _NCH)], dst_v)
    pltpu.sync_copy(zeros_hbm, acc.at[pl.ds(s * _RPT, _RPT)])
    plsc.subcore_barrier()

    def round_body(g, carry):
        gat = []
        for b in range(_RB):
            j = g * _RB + b

            def _drain(b=b, j=j):
                pltpu.make_async_copy(rows[b], acc.at[dst_v.at[j - _RB]],
                                      ssem[b]).wait()

            pl.when(g > 0)(_drain)
            wait_idx(j, b)
            gat.append(pltpu.async_copy(h_hbm.at[srcb[b].at[0]], rows[b],
                                        gsem[b]))
        for b in range(_RB):
            j = g * _RB + b
            gat[b].wait()
            pltpu.async_copy(rows[b], acc.at[dst_v.at[j]], ssem[b],
                             add=True)

            def _fire(b=b):
                fire_idx((g + 1) * _RB + b, b)

            pl.when(g + 1 < _NG)(_fire)
        return carry

    lax.fori_loop(0, _NG, round_body, 0)
    for b in range(_RB):
        pltpu.make_async_copy(rows[b], acc.at[dst_v.at[(_NG - 1) * _RB + b]],
                              ssem[b]).wait()

    plsc.subcore_barrier()
    pltpu.sync_copy(acc.at[pl.ds(s * _RPT, _RPT)],
                    out_hbm.at[pl.ds(c * _NP + s * _RPT, _RPT)])


_BLK = 1024


def _gru_body(g0_ref, g1_ref, h_ref, w_ref, wz_ref, uz_ref, wr_ref, ur_ref,
              wh_ref, uh_ref, out_ref):
    f32 = jnp.float32
    g = g0_ref[...] + g1_ref[...]
    h = h_ref[...]
    agg = jnp.dot(g, w_ref[...], preferred_element_type=f32)
    z = jax.nn.sigmoid(jnp.dot(agg, wz_ref[...], preferred_element_type=f32)
                       + jnp.dot(h, uz_ref[...], preferred_element_type=f32))
    r = jax.nn.sigmoid(jnp.dot(agg, wr_ref[...], preferred_element_type=f32)
                       + jnp.dot(h, ur_ref[...], preferred_element_type=f32))
    hh = jnp.tanh(jnp.dot(agg, wh_ref[...], preferred_element_type=f32)
                  + jnp.dot(r * h, uh_ref[...], preferred_element_type=f32))
    out_ref[...] = (1.0 - z) * h + z * hh


_row_spec = pl.BlockSpec((_BLK, _D), lambda i: (i, 0))
_w_spec = pl.BlockSpec((_D, _D), lambda i: (0, 0))

_gru = pl.pallas_call(
    _gru_body,
    grid=(_NP // _BLK,),
    in_specs=[_row_spec, _row_spec, _row_spec] + [_w_spec] * 7,
    out_specs=_row_spec,
    out_shape=jax.ShapeDtypeStruct((_NP, _D), jnp.float32),
)


def kernel(x, W, Wz, Uz, Wr, Ur, Wh, Uh, edge_index):
    src = edge_index[0].astype(jnp.int32).reshape(_NW * _NCH, _CH)
    dst = edge_index[1].astype(jnp.int32).reshape(_NW * _NCH, _CH)
    zeros = jnp.zeros((_RPT, _D), jnp.float32)
    h = jnp.pad(x, ((0, _NP - _N), (0, 0)))
    for l in range(_L):
        g2 = _segsum(h, src, dst, zeros)
        h = _gru(g2[:_NP], g2[_NP:], h, W[l], Wz, Uz, Wr, Ur, Wh, Uh)
    return jnp.concatenate([x, h[:_N]], axis=-1)
